# baseline (speedup 1.0000x reference)
import jax
import jax.numpy as jnp
from jax.experimental import pallas as pl
from jax.experimental.pallas import tpu as pltpu

B = 4
S = 2048
IN_F = 1024
OUT_F = 1024
NUM_BANKS = 16
NI = 2
SB = S // NI


def _body(sel_ref, bw_ref, x_ref, w0_ref, w1_ref, bias_ref, out_ref,
          weff_ref, beff_ref):
    b = pl.program_id(0)
    i = pl.program_id(1)

    @pl.when(i == 0)
    def _combine():
        bw0 = bw_ref[b, 0]
        bw1 = bw_ref[b, 1]
        weff_ref[...] = (bw0 * w0_ref[0] + bw1 * w1_ref[0]).astype(jnp.bfloat16)
        s0 = sel_ref[b, 0]
        s1 = sel_ref[b, 1]
        beff_ref[...] = (bw0 * bias_ref[s0, :] + bw1 * bias_ref[s1, :])[None, :]

    acc = jnp.dot(x_ref[0].astype(jnp.bfloat16), weff_ref[...],
                  preferred_element_type=jnp.float32)
    out_ref[0] = acc + beff_ref[...]


def kernel(tensor, bank_weights, bank_selections, W, bias):
    grid_spec = pltpu.PrefetchScalarGridSpec(
        num_scalar_prefetch=2,
        grid=(B, NI),
        in_specs=[
            pl.BlockSpec((1, SB, IN_F), lambda b, i, sel, bw: (b, i, 0)),
            pl.BlockSpec((1, IN_F, OUT_F), lambda b, i, sel, bw: (sel[b, 0], 0, 0)),
            pl.BlockSpec((1, IN_F, OUT_F), lambda b, i, sel, bw: (sel[b, 1], 0, 0)),
            pl.BlockSpec((NUM_BANKS, OUT_F), lambda b, i, sel, bw: (0, 0)),
        ],
        out_specs=pl.BlockSpec((1, SB, OUT_F), lambda b, i, sel, bw: (b, i, 0)),
        scratch_shapes=[
            pltpu.VMEM((IN_F, OUT_F), jnp.bfloat16),
            pltpu.VMEM((1, OUT_F), jnp.float32),
        ],
    )
    return pl.pallas_call(
        _body,
        grid_spec=grid_spec,
        out_shape=jax.ShapeDtypeStruct((B, S, OUT_F), jnp.float32),
    )(bank_selections, bank_weights, tensor, W, W, bias)
